# baseline (speedup 1.0000x reference)
import functools

import jax
import jax.numpy as jnp
from jax import lax
from jax.experimental import pallas as pl
from jax.experimental.pallas import tpu as pltpu
from jax.experimental.pallas import tpu_sc as plsc

RATIO = 0.1
ROWS = 128
FEAT = 32768
NWORK = 32
ROWS_PER_W = ROWS // NWORK
CHUNKS = FEAT // 16
INT_MIN = -2147483648


def _sc_kwta(x_hbm, out_hbm, buf_v, cand_v, hist_v, tot_v, off_s, sem, *, k):
    sid = lax.axis_index("s")
    wid = sid * 2 + lax.axis_index("c")
    lane = lax.iota(jnp.int32, 16)
    lane_base = lane * jnp.int32(256)
    ones16 = jnp.ones((16,), jnp.int32)

    def fold_and_scan(krem):
        @plsc.parallel_loop(0, 16, unroll=2)
        def fold_body(i):
            acc = hist_v[pl.ds(i * 16, 16)]
            hist_v[pl.ds(i * 16, 16)] = jnp.zeros((16,), jnp.int32)
            for l in range(1, 16):
                acc = acc + hist_v[pl.ds(l * 256 + i * 16, 16)]
                hist_v[pl.ds(l * 256 + i * 16, 16)] = jnp.zeros((16,),
                                                                jnp.int32)
            tot_v[pl.ds(i * 16, 16)] = acc

        def scan_body(i, carry):
            cum, chosen, kr = carry
            g = jnp.int32(15) - i
            v = tot_v[pl.ds(g * 16, 16)]
            rv = lax.rev(v, (0,))
            cs = plsc.cumsum(rv) + cum
            hit = cs >= kr
            npop = plsc.all_reduce_population_count(hit)
            p = plsc.all_reduce_ffs(hit)
            sel = lane == p
            cum_at = jnp.broadcast_to(
                jnp.sum(jnp.where(sel, cs, jnp.int32(0))), (16,))
            tot_at = jnp.broadcast_to(
                jnp.sum(jnp.where(sel, rv, jnp.int32(0))), (16,))
            found_now = jnp.logical_and(chosen < 0, npop > 0)
            chosen_local = g * 16 + (jnp.int32(15) - p)
            chosen = jnp.where(found_now, chosen_local, chosen)
            kr = jnp.where(found_now, kr - (cum_at - tot_at), kr)
            cum = jnp.broadcast_to(jnp.max(cs), (16,))
            return cum, chosen, kr

        _, chosen, krem = lax.fori_loop(
            0, 16, scan_body,
            (jnp.zeros((16,), jnp.int32),
             jnp.full((16,), -1, jnp.int32), krem))
        return chosen, krem

    @plsc.parallel_loop(0, 256, unroll=8)
    def zero_body(i):
        hist_v[pl.ds(i * 16, 16)] = jnp.zeros((16,), jnp.int32)

    for rr in range(ROWS_PER_W):
        row = wid * ROWS_PER_W + rr
        pltpu.sync_copy(x_hbm.at[row], buf_v)

        @plsc.parallel_loop(0, CHUNKS, unroll=8)
        def maphist_body(i):
            ci = buf_v[pl.ds(i * 16, 16)]
            ci = jnp.where(ci == jnp.int32(INT_MIN), jnp.int32(0), ci)
            m = ci ^ ((ci >> 31) & jnp.int32(0x7FFFFFFF))
            buf_v[pl.ds(i * 16, 16)] = m
            bucket = ((m >> 24) & jnp.int32(0xFF)) ^ jnp.int32(0x80)
            plsc.addupdate_scatter(hist_v, [lane_base + bucket], ones16)

        krem = jnp.full((16,), k, jnp.int32)
        chosen, krem = fold_and_scan(krem)
        prefix_val = (chosen ^ jnp.int32(0x80)) << 24
        prefix_mask = jnp.full((16,), INT_MIN >> 7, jnp.int32)

        off_s[0] = jnp.int32(0)
        pv1 = prefix_val
        pm1 = prefix_mask

        @plsc.parallel_loop(0, CHUNKS, unroll=8)
        def hist1_body(i, pv1=pv1, pm1=pm1):
            m = buf_v[pl.ds(i * 16, 16)]
            match = (m & pm1) == pv1
            bucket = (m >> 16) & jnp.int32(0xFF)
            plsc.addupdate_scatter(hist_v, [lane_base + bucket], ones16,
                                   mask=match)
            npop = plsc.all_reduce_population_count(match)
            off = plsc.fetch_and_add(off_s, npop[0], subcore_id=sid)
            plsc.store_compressed(cand_v.at[pl.ds(off, 16)], m, mask=match)

        chosen, krem = fold_and_scan(krem)
        prefix_val = prefix_val | (chosen << 16)
        prefix_mask = prefix_mask | jnp.full((16,), 0xFF << 16, jnp.int32)

        ncand = off_s[0]
        ngroups = (ncand + jnp.int32(15)) >> 4
        nc_v = jnp.broadcast_to(ncand, (16,))

        for t in range(2, 4):
            shift = 24 - 8 * t
            pv = prefix_val
            pm = prefix_mask

            @plsc.parallel_loop(0, ngroups, unroll=4)
            def histc_body(i, shift=shift, pv=pv, pm=pm):
                m = cand_v[pl.ds(i * 16, 16)]
                valid = (i * 16 + lane) < nc_v
                match = jnp.logical_and((m & pm) == pv, valid)
                bucket = (m >> shift) & jnp.int32(0xFF)
                plsc.addupdate_scatter(hist_v, [lane_base + bucket], ones16,
                                       mask=match)

            chosen, krem = fold_and_scan(krem)
            prefix_val = prefix_val | (chosen << shift)
            mask_c = (0xFF << shift) & 0xFFFFFFFF
            if mask_c >= 2**31:
                mask_c -= 2**32
            prefix_mask = prefix_mask | jnp.full((16,), mask_c, jnp.int32)

        thr_m = prefix_val

        @plsc.parallel_loop(0, CHUNKS, unroll=8)
        def mask_body(i):
            m = buf_v[pl.ds(i * 16, 16)]
            xi = m ^ ((m >> 31) & jnp.int32(0x7FFFFFFF))
            buf_v[pl.ds(i * 16, 16)] = jnp.where(m >= thr_m, xi,
                                                 jnp.int32(0))

        pltpu.sync_copy(buf_v, out_hbm.at[row])


def kernel(inputs):
    rows, features = inputs.shape
    k = max(int(round(RATIO * features)), 1)
    xi = lax.bitcast_convert_type(inputs, jnp.int32)
    mesh = plsc.VectorSubcoreMesh(core_axis_name="c", subcore_axis_name="s")
    out_i = pl.kernel(
        functools.partial(_sc_kwta, k=k),
        mesh=mesh,
        compiler_params=pltpu.CompilerParams(needs_layout_passes=False),
        out_type=jax.ShapeDtypeStruct((rows, features), jnp.int32),
        scratch_types=[
            pltpu.VMEM((FEAT,), jnp.int32),
            pltpu.VMEM((FEAT + 16,), jnp.int32),
            pltpu.VMEM((4096,), jnp.int32),
            pltpu.VMEM((256,), jnp.int32),
            pltpu.SMEM((1,), jnp.int32),
            pltpu.SemaphoreType.DMA,
        ],
    )(xi)
    return lax.bitcast_convert_type(out_i, jnp.float32)


# speedup vs baseline: 2.0946x; 2.0946x over previous
import functools

import jax
import jax.numpy as jnp
from jax import lax
from jax.experimental import pallas as pl
from jax.experimental.pallas import tpu as pltpu
from jax.experimental.pallas import tpu_sc as plsc

RATIO = 0.1
ROWS = 128
FEAT = 32768
NWORK = 32
ROWS_PER_W = ROWS // NWORK
CHUNKS = FEAT // 16
INT_MIN = -2147483648


def _sc_kwta(x_hbm, out_hbm, buf_v, hist_v, tot_v, sem, *, k):
    wid = lax.axis_index("s") * 2 + lax.axis_index("c")
    lane = lax.iota(jnp.int32, 16)
    lane_base = lane * jnp.int32(256)
    ones16 = jnp.ones((16,), jnp.int32)

    def fold_and_scan(krem):
        @plsc.parallel_loop(0, 16, unroll=2)
        def fold_body(i):
            acc = hist_v[pl.ds(i * 16, 16)]
            hist_v[pl.ds(i * 16, 16)] = jnp.zeros((16,), jnp.int32)
            for l in range(1, 16):
                acc = acc + hist_v[pl.ds(l * 256 + i * 16, 16)]
                hist_v[pl.ds(l * 256 + i * 16, 16)] = jnp.zeros((16,),
                                                                jnp.int32)
            tot_v[pl.ds(i * 16, 16)] = acc

        def scan_body(i, carry):
            cum, chosen, kr = carry
            g = jnp.int32(15) - i
            v = tot_v[pl.ds(g * 16, 16)]
            rv = lax.rev(v, (0,))
            cs = plsc.cumsum(rv) + cum
            hit = cs >= kr
            npop = plsc.all_reduce_population_count(hit)
            p = plsc.all_reduce_ffs(hit)
            sel = lane == p
            cum_at = jnp.broadcast_to(
                jnp.sum(jnp.where(sel, cs, jnp.int32(0))), (16,))
            tot_at = jnp.broadcast_to(
                jnp.sum(jnp.where(sel, rv, jnp.int32(0))), (16,))
            found_now = jnp.logical_and(chosen < 0, npop > 0)
            chosen_local = g * 16 + (jnp.int32(15) - p)
            chosen = jnp.where(found_now, chosen_local, chosen)
            kr = jnp.where(found_now, kr - (cum_at - tot_at), kr)
            cum = jnp.broadcast_to(jnp.max(cs), (16,))
            return cum, chosen, kr

        _, chosen, krem = lax.fori_loop(
            0, 16, scan_body,
            (jnp.zeros((16,), jnp.int32),
             jnp.full((16,), -1, jnp.int32), krem))
        return chosen, krem

    @plsc.parallel_loop(0, 256, unroll=8)
    def zero_body(i):
        hist_v[pl.ds(i * 16, 16)] = jnp.zeros((16,), jnp.int32)

    for rr in range(ROWS_PER_W):
        row = wid * ROWS_PER_W + rr
        pltpu.sync_copy(x_hbm.at[row], buf_v)

        @plsc.parallel_loop(0, CHUNKS, unroll=8)
        def maphist_body(i):
            ci = buf_v[pl.ds(i * 16, 16)]
            ci = jnp.where(ci == jnp.int32(INT_MIN), jnp.int32(0), ci)
            m = ci ^ ((ci >> 31) & jnp.int32(0x7FFFFFFF))
            buf_v[pl.ds(i * 16, 16)] = m
            bucket = ((m >> 24) & jnp.int32(0xFF)) ^ jnp.int32(0x80)
            plsc.addupdate_scatter(hist_v, [lane_base + bucket], ones16)

        krem = jnp.full((16,), k, jnp.int32)
        chosen, krem = fold_and_scan(krem)
        prefix_val = (chosen ^ jnp.int32(0x80)) << 24
        prefix_mask = jnp.full((16,), INT_MIN >> 7, jnp.int32)

        for t in range(1, 4):
            shift = 24 - 8 * t
            pv = prefix_val
            pm = prefix_mask

            @plsc.parallel_loop(0, CHUNKS, unroll=8)
            def hist_body(i, shift=shift, pv=pv, pm=pm):
                m = buf_v[pl.ds(i * 16, 16)]
                match = (m & pm) == pv
                bucket = (m >> shift) & jnp.int32(0xFF)
                plsc.addupdate_scatter(hist_v, [lane_base + bucket], ones16,
                                       mask=match)

            chosen, krem = fold_and_scan(krem)
            prefix_val = prefix_val | (chosen << shift)
            mask_c = (0xFF << shift) & 0xFFFFFFFF
            if mask_c >= 2**31:
                mask_c -= 2**32
            prefix_mask = prefix_mask | jnp.full((16,), mask_c, jnp.int32)

        thr_m = prefix_val

        @plsc.parallel_loop(0, CHUNKS, unroll=8)
        def mask_body(i):
            m = buf_v[pl.ds(i * 16, 16)]
            xi = m ^ ((m >> 31) & jnp.int32(0x7FFFFFFF))
            buf_v[pl.ds(i * 16, 16)] = jnp.where(m >= thr_m, xi,
                                                 jnp.int32(0))

        pltpu.sync_copy(buf_v, out_hbm.at[row])


def kernel(inputs):
    rows, features = inputs.shape
    k = max(int(round(RATIO * features)), 1)
    xi = lax.bitcast_convert_type(inputs, jnp.int32)
    mesh = plsc.VectorSubcoreMesh(core_axis_name="c", subcore_axis_name="s")
    out_i = pl.kernel(
        functools.partial(_sc_kwta, k=k),
        mesh=mesh,
        compiler_params=pltpu.CompilerParams(needs_layout_passes=False),
        out_type=jax.ShapeDtypeStruct((rows, features), jnp.int32),
        scratch_types=[
            pltpu.VMEM((FEAT,), jnp.int32),
            pltpu.VMEM((4096,), jnp.int32),
            pltpu.VMEM((256,), jnp.int32),
            pltpu.SemaphoreType.DMA,
        ],
    )(xi)
    return lax.bitcast_convert_type(out_i, jnp.float32)
